# baseline (speedup 1.0000x reference)
import functools

import jax
import jax.numpy as jnp
from jax import lax
from jax.experimental import pallas as pl
from jax.experimental.pallas import tpu as pltpu
from jax.experimental.pallas import tpu_sc as plsc

VOCAB = 32000
D = 512
B = 1024
L = 200

DW = D // 2

NC = 2
NS = 16
NW = NC * NS
ROWS_PER_W = B // NW
CHUNK = 40
NCHUNK = L // CHUNK
NCH = DW // 16
GROUP = 4


BPACK = 1280


def _rne_bf16_bits(bits):
    return bits + jnp.int32(0x7FFF) + ((bits >> 16) & jnp.int32(1))


def _pack_body(x_ref, o_ref):
    lo_bits = jax.lax.bitcast_convert_type(x_ref[:, :DW], jnp.int32)
    hi_bits = jax.lax.bitcast_convert_type(x_ref[:, DW:], jnp.int32)
    lo = (_rne_bf16_bits(lo_bits) >> 16) & jnp.int32(0xFFFF)
    hi = _rne_bf16_bits(hi_bits) & jnp.int32(-65536)
    o_ref[...] = lo | hi


def _pack_table(table):
    return pl.pallas_call(
        _pack_body,
        grid=(VOCAB // BPACK,),
        in_specs=[pl.BlockSpec((BPACK, D), lambda i: (i, 0))],
        out_specs=pl.BlockSpec((BPACK, DW), lambda i: (i, 0)),
        out_shape=jax.ShapeDtypeStruct((VOCAB, DW), jnp.int32),
    )(table)


_mesh = plsc.VectorSubcoreMesh(core_axis_name="c", subcore_axis_name="s")


def _make_pool(rows):
    rpw = rows // NW

    @functools.partial(
        pl.kernel,
        mesh=_mesh,
        out_type=jax.ShapeDtypeStruct((rows, D), jnp.float32),
        scratch_types=[
            pltpu.VMEM((rpw * L,), jnp.int32),
            pltpu.VMEM((CHUNK, DW), jnp.int32),
            pltpu.VMEM((CHUNK, DW), jnp.int32),
            pltpu.VMEM((rpw, D), jnp.float32),
            pltpu.SemaphoreType.DMA,
            pltpu.SemaphoreType.DMA,
        ],
    )
    def _pool(ids_hbm, table_hbm, out_hbm, ids_v, g0, g1, outb, sem0, sem1):
        wid = lax.axis_index("s") * NC + lax.axis_index("c")
        base = wid * rpw
        pltpu.sync_copy(ids_hbm.at[pl.ds(base * L, rpw * L)], ids_v)

        gbufs = (g0, g1)
        sems = (sem0, sem1)

        def _accum_step(g, l, a):
            out = []
            for i in range(NCH):
                w = g[l, pl.ds(i * 16, 16)]
                lo = lax.bitcast_convert_type(w << 16, jnp.float32)
                hi = lax.bitcast_convert_type(w, jnp.float32)
                out.append(a[2 * i] + lo)
                out.append(a[2 * i + 1] + hi)
            return tuple(out)

        def group_body(gi, carry):
            r0 = gi * GROUP
            handles = [None, None]
            handles[0] = pltpu.async_copy(
                table_hbm.at[ids_v.at[pl.ds(r0 * L, CHUNK)]], g0, sem0)
            for rr in range(GROUP):
                accs = tuple(
                    jnp.zeros((16,), jnp.float32) for _ in range(2 * NCH))
                for k in range(NCHUNK):
                    gidx = rr * NCHUNK + k
                    if gidx + 1 < GROUP * NCHUNK:
                        handles[(gidx + 1) % 2] = pltpu.async_copy(
                            table_hbm.at[
                                ids_v.at[
                                    pl.ds(r0 * L + (gidx + 1) * CHUNK, CHUNK)]],
                            gbufs[(gidx + 1) % 2], sems[(gidx + 1) % 2])
                    handles[gidx % 2].wait()
                    g = gbufs[gidx % 2]

                    def chunk_body(l, a, g=g):
                        return _accum_step(g, 2 * l + 1, _accum_step(g, 2 * l, a))

                    accs = lax.fori_loop(0, CHUNK // 2, chunk_body, accs)
                r = r0 + rr
                for i in range(NCH):
                    outb[r, pl.ds(i * 16, 16)] = accs[2 * i]
                    outb[r, pl.ds(DW + i * 16, 16)] = accs[2 * i + 1]
            return carry

        lax.fori_loop(0, rpw // GROUP, group_body, 0)
        pltpu.sync_copy(outb, out_hbm.at[pl.ds(base, rpw)])

    return _pool


BV = 1280


def _mm_body(*refs):
    p_ref, w_ref, b_ref = refs[0], refs[1], refs[2]
    o_ref = refs[-1]
    p = p_ref[...].astype(jnp.bfloat16)
    wp = w_ref[...]
    wlo = jax.lax.bitcast_convert_type(wp << 16, jnp.float32)
    whi = jax.lax.bitcast_convert_type(wp & jnp.int32(-65536), jnp.float32)
    dn = (((1,), (1,)), ((), ()))
    acc = jax.lax.dot_general(
        p[:, :DW], wlo.astype(jnp.bfloat16), dn,
        preferred_element_type=jnp.float32)
    acc += jax.lax.dot_general(
        p[:, DW:], whi.astype(jnp.bfloat16), dn,
        preferred_element_type=jnp.float32)
    o_ref[...] = acc * jnp.float32(1.0 / L) + b_ref[...]


def _project_chunk(sums_c, W, b2d, row_block, o_prev):
    bc = sums_c.shape[0]
    ins = [sums_c, W, b2d]
    in_specs = [
        pl.BlockSpec((bc, D), lambda i: (0, 0)),
        pl.BlockSpec((BV, DW), lambda i: (i, 0)),
        pl.BlockSpec((1, BV), lambda i: (0, i)),
    ]
    kwargs = {}
    if o_prev is not None:
        ins.append(o_prev)
        in_specs.append(pl.BlockSpec(memory_space=pl.ANY))
        kwargs["input_output_aliases"] = {3: 0}
    return pl.pallas_call(
        _mm_body,
        grid=(VOCAB // BV,),
        in_specs=in_specs,
        out_specs=pl.BlockSpec((bc, BV), lambda i: (row_block, i)),
        out_shape=jax.ShapeDtypeStruct((B, VOCAB), jnp.float32),
        **kwargs,
    )(*ins)


NCB = 2
BC = B // NCB


def kernel(input_ids, emb_table, W, b):
    ids = input_ids.astype(jnp.int32).reshape(B * L)
    table_packed = _pack_table(emb_table)
    pool_fn = _make_pool(BC)
    b2d = b.reshape(1, VOCAB)
    sums = [pool_fn(ids[c * BC * L:(c + 1) * BC * L], table_packed)
            for c in range(NCB)]
    w_packed = _pack_table(W)
    out = None
    for c in range(NCB):
        out = _project_chunk(sums[c], w_packed, b2d, c, out)
    return out


# speedup vs baseline: 1.0771x; 1.0771x over previous
import functools

import jax
import jax.numpy as jnp
from jax import lax
from jax.experimental import pallas as pl
from jax.experimental.pallas import tpu as pltpu
from jax.experimental.pallas import tpu_sc as plsc

VOCAB = 32000
D = 512
B = 1024
L = 200

DW = D // 2

NC = 2
NS = 16
NW = NC * NS
ROWS_PER_W = B // NW
CHUNK = 40
NCHUNK = L // CHUNK
NCH = DW // 16
GROUP = 4
NBUF = 3
DEPTH = 2


BPACK = 1280


def _rne_bf16_bits(bits):
    return bits + jnp.int32(0x7FFF) + ((bits >> 16) & jnp.int32(1))


def _pack_body(x_ref, o_ref):
    lo_bits = jax.lax.bitcast_convert_type(x_ref[:, :DW], jnp.int32)
    hi_bits = jax.lax.bitcast_convert_type(x_ref[:, DW:], jnp.int32)
    lo = (_rne_bf16_bits(lo_bits) >> 16) & jnp.int32(0xFFFF)
    hi = _rne_bf16_bits(hi_bits) & jnp.int32(-65536)
    o_ref[...] = lo | hi


def _pack_table(table):
    return pl.pallas_call(
        _pack_body,
        grid=(VOCAB // BPACK,),
        in_specs=[pl.BlockSpec((BPACK, D), lambda i: (i, 0))],
        out_specs=pl.BlockSpec((BPACK, DW), lambda i: (i, 0)),
        out_shape=jax.ShapeDtypeStruct((VOCAB, DW), jnp.int32),
    )(table)


_mesh = plsc.VectorSubcoreMesh(core_axis_name="c", subcore_axis_name="s")


def _make_pool(rows):
    rpw = rows // NW

    @functools.partial(
        pl.kernel,
        mesh=_mesh,
        out_type=jax.ShapeDtypeStruct((rows, D), jnp.float32),
        scratch_types=[
            pltpu.VMEM((rpw * L,), jnp.int32),
            pltpu.VMEM((CHUNK, DW), jnp.int32),
            pltpu.VMEM((CHUNK, DW), jnp.int32),
            pltpu.VMEM((CHUNK, DW), jnp.int32),
            pltpu.VMEM((rpw, D), jnp.float32),
            pltpu.SemaphoreType.DMA,
            pltpu.SemaphoreType.DMA,
            pltpu.SemaphoreType.DMA,
        ],
    )
    def _pool(ids_hbm, table_hbm, out_hbm, ids_v, g0, g1, g2, outb,
              sem0, sem1, sem2):
        wid = lax.axis_index("s") * NC + lax.axis_index("c")
        base = wid * rpw
        pltpu.sync_copy(ids_hbm.at[pl.ds(base * L, rpw * L)], ids_v)

        gbufs = (g0, g1, g2)
        sems = (sem0, sem1, sem2)

        def _accum_step(g, l, a):
            out = []
            for i in range(NCH):
                w = g[l, pl.ds(i * 16, 16)]
                lo = lax.bitcast_convert_type(w << 16, jnp.float32)
                hi = lax.bitcast_convert_type(w, jnp.float32)
                out.append(a[2 * i] + lo)
                out.append(a[2 * i + 1] + hi)
            return tuple(out)

        total = GROUP * NCHUNK

        def group_body(gi, carry):
            r0 = gi * GROUP

            def issue(gidx):
                return pltpu.async_copy(
                    table_hbm.at[
                        ids_v.at[pl.ds(r0 * L + gidx * CHUNK, CHUNK)]],
                    gbufs[gidx % NBUF], sems[gidx % NBUF])

            handles = {}
            for gidx in range(DEPTH):
                handles[gidx] = issue(gidx)
            for rr in range(GROUP):
                accs = tuple(
                    jnp.zeros((16,), jnp.float32) for _ in range(2 * NCH))
                for k in range(NCHUNK):
                    gidx = rr * NCHUNK + k
                    if gidx + DEPTH < total:
                        handles[gidx + DEPTH] = issue(gidx + DEPTH)
                    handles[gidx].wait()
                    g = gbufs[gidx % NBUF]

                    def chunk_body(l, a, g=g):
                        return _accum_step(g, 2 * l + 1, _accum_step(g, 2 * l, a))

                    accs = lax.fori_loop(0, CHUNK // 2, chunk_body, accs)
                r = r0 + rr
                for i in range(NCH):
                    outb[r, pl.ds(i * 16, 16)] = accs[2 * i]
                    outb[r, pl.ds(DW + i * 16, 16)] = accs[2 * i + 1]
            return carry

        lax.fori_loop(0, rpw // GROUP, group_body, 0)
        pltpu.sync_copy(outb, out_hbm.at[pl.ds(base, rpw)])

    return _pool


BV = 1280


def _mm_body(*refs):
    p_ref, w_ref, b_ref = refs[0], refs[1], refs[2]
    o_ref = refs[-1]
    p = p_ref[...].astype(jnp.bfloat16)
    wp = w_ref[...]
    wlo = jax.lax.bitcast_convert_type(wp << 16, jnp.float32)
    whi = jax.lax.bitcast_convert_type(wp & jnp.int32(-65536), jnp.float32)
    dn = (((1,), (1,)), ((), ()))
    acc = jax.lax.dot_general(
        p[:, :DW], wlo.astype(jnp.bfloat16), dn,
        preferred_element_type=jnp.float32)
    acc += jax.lax.dot_general(
        p[:, DW:], whi.astype(jnp.bfloat16), dn,
        preferred_element_type=jnp.float32)
    o_ref[...] = acc * jnp.float32(1.0 / L) + b_ref[...]


def _project_chunk(sums_c, W, b2d, row_block, o_prev):
    bc = sums_c.shape[0]
    ins = [sums_c, W, b2d]
    in_specs = [
        pl.BlockSpec((bc, D), lambda i: (0, 0)),
        pl.BlockSpec((BV, DW), lambda i: (i, 0)),
        pl.BlockSpec((1, BV), lambda i: (0, i)),
    ]
    kwargs = {}
    if o_prev is not None:
        ins.append(o_prev)
        in_specs.append(pl.BlockSpec(memory_space=pl.ANY))
        kwargs["input_output_aliases"] = {3: 0}
    return pl.pallas_call(
        _mm_body,
        grid=(VOCAB // BV,),
        in_specs=in_specs,
        out_specs=pl.BlockSpec((bc, BV), lambda i: (row_block, i)),
        out_shape=jax.ShapeDtypeStruct((B, VOCAB), jnp.float32),
        **kwargs,
    )(*ins)


NCB = 2
BC = B // NCB


def kernel(input_ids, emb_table, W, b):
    ids = input_ids.astype(jnp.int32).reshape(B * L)
    table_packed = _pack_table(emb_table)
    pool_fn = _make_pool(BC)
    b2d = b.reshape(1, VOCAB)
    sums = [pool_fn(ids[c * BC * L:(c + 1) * BC * L], table_packed)
            for c in range(NCB)]
    w_packed = _pack_table(W)
    out = None
    for c in range(NCB):
        out = _project_chunk(sums[c], w_packed, b2d, c, out)
    return out


# speedup vs baseline: 1.1075x; 1.0282x over previous
import functools

import jax
import jax.numpy as jnp
from jax import lax
from jax.experimental import pallas as pl
from jax.experimental.pallas import tpu as pltpu
from jax.experimental.pallas import tpu_sc as plsc

VOCAB = 32000
D = 512
B = 1024
L = 200

DW = D // 2

NC = 2
NS = 16
NW = NC * NS
ROWS_PER_W = B // NW
CHUNK = 40
NCHUNK = L // CHUNK
NCH = DW // 16
GROUP = 8
NBUF = 4
DEPTH = 3


BPACK = 1280


def _rne_bf16_bits(bits):
    return bits + jnp.int32(0x7FFF) + ((bits >> 16) & jnp.int32(1))


def _pack_body(x_ref, o_ref):
    lo_bits = jax.lax.bitcast_convert_type(x_ref[:, :DW], jnp.int32)
    hi_bits = jax.lax.bitcast_convert_type(x_ref[:, DW:], jnp.int32)
    lo = (_rne_bf16_bits(lo_bits) >> 16) & jnp.int32(0xFFFF)
    hi = _rne_bf16_bits(hi_bits) & jnp.int32(-65536)
    o_ref[...] = lo | hi


def _pack_table(table):
    return pl.pallas_call(
        _pack_body,
        grid=(VOCAB // BPACK,),
        in_specs=[pl.BlockSpec((BPACK, D), lambda i: (i, 0))],
        out_specs=pl.BlockSpec((BPACK, DW), lambda i: (i, 0)),
        out_shape=jax.ShapeDtypeStruct((VOCAB, DW), jnp.int32),
    )(table)


_mesh = plsc.VectorSubcoreMesh(core_axis_name="c", subcore_axis_name="s")


def _make_pool(rows):
    rpw = rows // NW

    @functools.partial(
        pl.kernel,
        mesh=_mesh,
        out_type=jax.ShapeDtypeStruct((rows, D), jnp.float32),
        scratch_types=[
            pltpu.VMEM((rpw * L,), jnp.int32),
            pltpu.VMEM((CHUNK, DW), jnp.int32),
            pltpu.VMEM((CHUNK, DW), jnp.int32),
            pltpu.VMEM((CHUNK, DW), jnp.int32),
            pltpu.VMEM((CHUNK, DW), jnp.int32),
            pltpu.VMEM((rpw, D), jnp.float32),
            pltpu.SemaphoreType.DMA,
            pltpu.SemaphoreType.DMA,
            pltpu.SemaphoreType.DMA,
            pltpu.SemaphoreType.DMA,
        ],
    )
    def _pool(ids_hbm, table_hbm, out_hbm, ids_v, g0, g1, g2, g3, outb,
              sem0, sem1, sem2, sem3):
        wid = lax.axis_index("s") * NC + lax.axis_index("c")
        base = wid * rpw
        pltpu.sync_copy(ids_hbm.at[pl.ds(base * L, rpw * L)], ids_v)

        gbufs = (g0, g1, g2, g3)
        sems = (sem0, sem1, sem2, sem3)

        def _accum_step(g, l, a):
            out = []
            for i in range(NCH):
                w = g[l, pl.ds(i * 16, 16)]
                lo = lax.bitcast_convert_type(w << 16, jnp.float32)
                hi = lax.bitcast_convert_type(w, jnp.float32)
                out.append(a[2 * i] + lo)
                out.append(a[2 * i + 1] + hi)
            return tuple(out)

        total = GROUP * NCHUNK

        def group_body(gi, carry):
            r0 = gi * GROUP

            def issue(gidx):
                return pltpu.async_copy(
                    table_hbm.at[
                        ids_v.at[pl.ds(r0 * L + gidx * CHUNK, CHUNK)]],
                    gbufs[gidx % NBUF], sems[gidx % NBUF])

            handles = {}
            for gidx in range(DEPTH):
                handles[gidx] = issue(gidx)
            for rr in range(GROUP):
                accs = tuple(
                    jnp.zeros((16,), jnp.float32) for _ in range(2 * NCH))
                for k in range(NCHUNK):
                    gidx = rr * NCHUNK + k
                    if gidx + DEPTH < total:
                        handles[gidx + DEPTH] = issue(gidx + DEPTH)
                    handles[gidx].wait()
                    g = gbufs[gidx % NBUF]

                    def chunk_body(l, a, g=g):
                        return _accum_step(g, 2 * l + 1, _accum_step(g, 2 * l, a))

                    accs = lax.fori_loop(0, CHUNK // 2, chunk_body, accs)
                r = r0 + rr
                for i in range(NCH):
                    outb[r, pl.ds(i * 16, 16)] = accs[2 * i]
                    outb[r, pl.ds(DW + i * 16, 16)] = accs[2 * i + 1]
            return carry

        lax.fori_loop(0, rpw // GROUP, group_body, 0)
        pltpu.sync_copy(outb, out_hbm.at[pl.ds(base, rpw)])

    return _pool


BV = 1280


def _mm_body(*refs):
    p_ref, w_ref, b_ref = refs[0], refs[1], refs[2]
    o_ref = refs[-1]
    p = p_ref[...].astype(jnp.bfloat16)
    wp = w_ref[...]
    wlo = jax.lax.bitcast_convert_type(wp << 16, jnp.float32)
    whi = jax.lax.bitcast_convert_type(wp & jnp.int32(-65536), jnp.float32)
    dn = (((1,), (1,)), ((), ()))
    acc = jax.lax.dot_general(
        p[:, :DW], wlo.astype(jnp.bfloat16), dn,
        preferred_element_type=jnp.float32)
    acc += jax.lax.dot_general(
        p[:, DW:], whi.astype(jnp.bfloat16), dn,
        preferred_element_type=jnp.float32)
    o_ref[...] = acc * jnp.float32(1.0 / L) + b_ref[...]


def _project_chunk(sums_c, W, b2d, row_block, o_prev):
    bc = sums_c.shape[0]
    ins = [sums_c, W, b2d]
    in_specs = [
        pl.BlockSpec((bc, D), lambda i: (0, 0)),
        pl.BlockSpec((BV, DW), lambda i: (i, 0)),
        pl.BlockSpec((1, BV), lambda i: (0, i)),
    ]
    kwargs = {}
    if o_prev is not None:
        ins.append(o_prev)
        in_specs.append(pl.BlockSpec(memory_space=pl.ANY))
        kwargs["input_output_aliases"] = {3: 0}
    return pl.pallas_call(
        _mm_body,
        grid=(VOCAB // BV,),
        in_specs=in_specs,
        out_specs=pl.BlockSpec((bc, BV), lambda i: (row_block, i)),
        out_shape=jax.ShapeDtypeStruct((B, VOCAB), jnp.float32),
        **kwargs,
    )(*ins)


NCB = 2
BC = B // NCB


def kernel(input_ids, emb_table, W, b):
    ids = input_ids.astype(jnp.int32).reshape(B * L)
    table_packed = _pack_table(emb_table)
    pool_fn = _make_pool(BC)
    b2d = b.reshape(1, VOCAB)
    sums = [pool_fn(ids[c * BC * L:(c + 1) * BC * L], table_packed)
            for c in range(NCB)]
    w_packed = _pack_table(W)
    out = None
    for c in range(NCB):
        out = _project_chunk(sums[c], w_packed, b2d, c, out)
    return out
